# Initial kernel scaffold; baseline (speedup 1.0000x reference)
import jax
import jax.numpy as jnp
from jax.experimental import pallas as pl


def kernel(x, edge_index, batch, W1, b1, Wr1, br1, Wc0, bc0, Wc1, bc1, Wr, br, Wl1, bl1, Wl2, bl2):
    raise NotImplementedError("write your pallas kernel here")


# speedup vs baseline: 16.0097x; 16.0097x over previous
import functools

import jax
import jax.numpy as jnp
from jax import lax
from jax.experimental import pallas as pl
from jax.experimental.pallas import tpu as pltpu
from jax.experimental.pallas import tpu_sc as plsc

NC = 2
NS = 16
NW = NC * NS
CH = 128
G = 64


def _sc_degree(dsts3, consts16, acc_rows):
    nchunk = dsts3.shape[1]
    rpt = acc_rows // NS
    zrep = rpt // CH
    mesh = plsc.VectorSubcoreMesh(core_axis_name="c", subcore_axis_name="s")

    @functools.partial(
        pl.kernel,
        out_type=jax.ShapeDtypeStruct((NC, acc_rows, 16), jnp.float32),
        mesh=mesh,
        scratch_types=[
            pltpu.VMEM((nchunk, CH), jnp.int32),
            pltpu.VMEM((CH, 16), jnp.float32),
            pltpu.VMEM((CH, 16), jnp.float32),
            pltpu.VMEM_SHARED((acc_rows, 16), jnp.float32),
        ],
    )
    def deg_kernel(dsts_hbm, consts_hbm, out_hbm, idx_d, zv, ov, acc):
        c = lax.axis_index("c")
        s = lax.axis_index("s")
        w = s * NC + c
        pltpu.sync_copy(dsts_hbm.at[w], idx_d)
        pltpu.sync_copy(consts_hbm.at[0], zv)
        pltpu.sync_copy(consts_hbm.at[1], ov)
        base = s * rpt
        for r in range(zrep):
            pltpu.sync_copy(zv, acc.at[pl.ds(base + r * CH, CH)])
        plsc.subcore_barrier()

        def body(j, carry):
            pltpu.sync_copy(ov, acc.at[idx_d.at[j]], add=True)
            return carry

        lax.fori_loop(0, nchunk, body, 0)
        plsc.subcore_barrier()
        pltpu.sync_copy(acc.at[pl.ds(base, rpt)],
                        out_hbm.at[c, pl.ds(base, rpt)])

    return deg_kernel(dsts3, consts16)


def _sc_propagate(y, srcs3, dsts3, zblk, acc_rows):
    d = y.shape[1]
    nchunk = srcs3.shape[1]
    rpt = acc_rows // NS
    zrep = rpt // CH
    mesh = plsc.VectorSubcoreMesh(core_axis_name="c", subcore_axis_name="s")

    @functools.partial(
        pl.kernel,
        out_type=jax.ShapeDtypeStruct((NC, acc_rows, d), jnp.float32),
        mesh=mesh,
        scratch_types=[
            pltpu.VMEM((nchunk, CH), jnp.int32),
            pltpu.VMEM((nchunk, CH), jnp.int32),
            pltpu.VMEM((CH, d), jnp.float32),
            pltpu.VMEM_SHARED((acc_rows, d), jnp.float32),
            pltpu.SemaphoreType.DMA,
        ],
    )
    def prop_kernel(y_hbm, srcs_hbm, dsts_hbm, zblk_hbm, out_hbm,
                    idx_s, idx_d, rows, acc, sem):
        c = lax.axis_index("c")
        s = lax.axis_index("s")
        w = s * NC + c
        pltpu.sync_copy(srcs_hbm.at[w], idx_s)
        pltpu.sync_copy(dsts_hbm.at[w], idx_d)
        pltpu.sync_copy(zblk_hbm, rows)
        base = s * rpt
        for r in range(zrep):
            pltpu.sync_copy(rows, acc.at[pl.ds(base + r * CH, CH)])
        plsc.subcore_barrier()

        def body(j, carry):
            pltpu.async_copy(y_hbm.at[idx_s.at[j]], rows, sem).wait()
            pltpu.sync_copy(rows, acc.at[idx_d.at[j]], add=True)
            return carry

        lax.fori_loop(0, nchunk, body, 0)
        plsc.subcore_barrier()
        pltpu.sync_copy(acc.at[pl.ds(base, rpt)],
                        out_hbm.at[c, pl.ds(base, rpt)])

    return prop_kernel(y, srcs3, dsts3, zblk)


def _tc_prep(deg2, x, W1, Wr1):
    n = x.shape[0]
    h = W1.shape[1]

    def body(deg_ref, x_ref, w_ref, wr_ref, y_ref):
        deg = deg_ref[0, :n, 0:1] + deg_ref[1, :n, 0:1] + 1.0
        dis = lax.rsqrt(deg)
        w = 0.95 * w_ref[...] + 0.05 * wr_ref[...]
        y_ref[...] = jnp.dot(x_ref[...], w,
                             preferred_element_type=jnp.float32) * dis

    return pl.pallas_call(
        body, out_shape=jax.ShapeDtypeStruct((n, h), jnp.float32),
    )(deg2, x, W1, Wr1)


def _tc_mid(acc2, y_prev, deg2, bc, br, Wc, Wr):
    n, h = y_prev.shape

    def body(acc_ref, y_ref, deg_ref, bc_ref, br_ref, wc_ref, wr_ref, o_ref):
        deg = deg_ref[0, :n, 0:1] + deg_ref[1, :n, 0:1] + 1.0
        dis = lax.rsqrt(deg)
        b = 0.95 * bc_ref[...] + 0.05 * br_ref[...]
        tot = acc_ref[0, :n, :] + acc_ref[1, :n, :] + y_ref[...]
        hh = jnp.maximum(tot * dis + b, 0.0)
        w = 0.95 * wc_ref[...] + 0.05 * wr_ref[...]
        o_ref[...] = jnp.dot(hh, w, preferred_element_type=jnp.float32) * dis

    return pl.pallas_call(
        body, out_shape=jax.ShapeDtypeStruct((n, h), jnp.float32),
    )(acc2, y_prev, deg2, bc, br, Wc, Wr)


def _tc_final(acc2, y3, deg2, bc, br, batch2d, Wl1, bl1, Wl2, bl2):
    n, h = y3.shape
    c_out = Wl2.shape[1]

    def body(acc_ref, y_ref, deg_ref, bc_ref, br_ref, bat_ref,
             wl1_ref, bl1_ref, wl2_ref, bl2_ref, o_ref):
        deg = deg_ref[0, :n, 0:1] + deg_ref[1, :n, 0:1] + 1.0
        dis = lax.rsqrt(deg)
        b = 0.95 * bc_ref[...] + 0.05 * br_ref[...]
        tot = acc_ref[0, :n, :] + acc_ref[1, :n, :] + y_ref[...]
        hh = jnp.maximum(tot * dis + b, 0.0)
        gid = lax.broadcasted_iota(jnp.int32, (G, n), 0)
        onehot = jnp.where(gid == jnp.broadcast_to(bat_ref[...], (G, n)),
                           1.0, 0.0)
        sums = jnp.dot(onehot, hh, preferred_element_type=jnp.float32)
        counts = jnp.sum(onehot, axis=1, keepdims=True)
        pooled = sums / jnp.maximum(counts, 1.0)
        z = jnp.maximum(
            jnp.dot(pooled, wl1_ref[...],
                    preferred_element_type=jnp.float32) + bl1_ref[...], 0.0)
        z = jnp.dot(z, wl2_ref[...],
                    preferred_element_type=jnp.float32) + bl2_ref[...]
        m = jnp.max(z, axis=1, keepdims=True)
        lse = jnp.log(jnp.sum(jnp.exp(z - m), axis=1, keepdims=True)) + m
        o_ref[...] = z - lse

    return pl.pallas_call(
        body, out_shape=jax.ShapeDtypeStruct((G, c_out), jnp.float32),
    )(acc2, y3, deg2, bc, br, batch2d, Wl1, bl1, Wl2, bl2)


def kernel(x, edge_index, batch, W1, b1, Wr1, br1, Wc0, bc0, Wc1, bc1,
           Wr, br, Wl1, bl1, Wl2, bl2):
    n, d = x.shape
    e = edge_index.shape[1]
    h = W1.shape[1]

    rpt = -(-n // (NS * CH)) * CH
    acc_rows = NS * rpt
    nchunk = -(-e // (NW * CH))
    e_pad = NW * nchunk * CH - e

    srcs = edge_index[0]
    dsts = edge_index[1]
    if e_pad:
        srcs = jnp.concatenate([srcs, jnp.zeros((e_pad,), jnp.int32)])
        dsts = jnp.concatenate([dsts, jnp.full((e_pad,), n, jnp.int32)])
    srcs3 = srcs.reshape(NW, nchunk, CH)
    dsts3 = dsts.reshape(NW, nchunk, CH)

    zblk = jnp.zeros((CH, d), jnp.float32)
    consts16 = jnp.stack([jnp.zeros((CH, 16), jnp.float32),
                          jnp.ones((CH, 16), jnp.float32)])
    batch2d = batch.reshape(1, n)
    b1r, br1r = b1.reshape(1, h), br1.reshape(1, h)
    bc0r, bc1r, brr = bc0.reshape(1, h), bc1.reshape(1, h), br.reshape(1, h)
    bl1r = bl1.reshape(1, h)
    bl2r = bl2.reshape(1, Wl2.shape[1])

    deg2 = _sc_degree(dsts3, consts16, acc_rows)
    y1 = _tc_prep(deg2, x, W1, Wr1)
    acc = _sc_propagate(y1, srcs3, dsts3, zblk, acc_rows)
    y2 = _tc_mid(acc, y1, deg2, b1r, br1r, Wc0, Wr)
    acc = _sc_propagate(y2, srcs3, dsts3, zblk, acc_rows)
    y3 = _tc_mid(acc, y2, deg2, bc0r, brr, Wc1, Wr)
    acc = _sc_propagate(y3, srcs3, dsts3, zblk, acc_rows)
    return _tc_final(acc, y3, deg2, bc1r, brr, batch2d, Wl1, bl1r, Wl2, bl2r)
